# baseline (speedup 1.0000x reference)
import jax
import jax.numpy as jnp
from jax.experimental import pallas as pl
from jax.experimental.pallas import tpu as pltpu

B, C, H, W = 64, 384, 32, 32


def _copy_body(idx_ref, x_ref, o_ref):
    o_ref[...] = x_ref[...]


def kernel(x, indices):
    idx = indices.astype(jnp.int32)
    grid_spec = pltpu.PrefetchScalarGridSpec(
        num_scalar_prefetch=1,
        grid=(C,),
        in_specs=[
            pl.BlockSpec((B, 1, H, W), lambda i, idx_ref: (0, idx_ref[i], 0, 0)),
        ],
        out_specs=pl.BlockSpec((B, 1, H, W), lambda i, idx_ref: (0, i, 0, 0)),
    )
    return pl.pallas_call(
        _copy_body,
        grid_spec=grid_spec,
        out_shape=jax.ShapeDtypeStruct((B, C, H, W), x.dtype),
    )(idx, x)


# speedup vs baseline: 1.6598x; 1.6598x over previous
import functools

import jax
import jax.numpy as jnp
from jax import lax
from jax.experimental import pallas as pl
from jax.experimental.pallas import tpu as pltpu
from jax.experimental.pallas import tpu_sc as plsc

B, C, H, W = 64, 384, 32, 32
D = H * W
NC, NS, L = 2, 16, 16
NW = NC * NS
BPW = B // NW
K = 32
NCHUNK = C // K


def _sc_body(x_hbm, idx_hbm, out_hbm, idx_v, adj_v, buf, sem):
    wid = lax.axis_index("s") * NC + lax.axis_index("c")
    pltpu.sync_copy(idx_hbm, idx_v)
    for bi in range(BPW):
        base = (wid * BPW + bi) * C
        for t in range(C // L):
            sl = pl.ds(t * L, L)
            adj_v[sl] = idx_v[sl] + base
        for j in range(NCHUNK):
            pltpu.async_copy(
                x_hbm.at[adj_v.at[pl.ds(j * K, K)]], buf, sem
            ).wait()
            pltpu.sync_copy(buf, out_hbm.at[pl.ds(base + j * K, K)])


@functools.partial(jax.jit, donate_argnums=())
def _sc_gather(x2, idx):
    mesh = plsc.VectorSubcoreMesh(core_axis_name="c", subcore_axis_name="s")
    return pl.kernel(
        _sc_body,
        mesh=mesh,
        out_type=jax.ShapeDtypeStruct((B * C, D), jnp.float32),
        scratch_types=[
            pltpu.VMEM((C,), jnp.int32),
            pltpu.VMEM((C,), jnp.int32),
            pltpu.VMEM((K, D), jnp.float32),
            pltpu.SemaphoreType.DMA,
        ],
    )(x2, idx)


def kernel(x, indices):
    idx = indices.astype(jnp.int32)
    out2 = _sc_gather(x.reshape(B * C, D), idx)
    return out2.reshape(B, C, H, W)


# speedup vs baseline: 1.7062x; 1.0280x over previous
import functools

import jax
import jax.numpy as jnp
from jax import lax
from jax.experimental import pallas as pl
from jax.experimental.pallas import tpu as pltpu
from jax.experimental.pallas import tpu_sc as plsc

B, C, H, W = 64, 384, 32, 32
D = H * W
NC, NS, L = 2, 16, 16
NW = NC * NS
BPW = B // NW
RPW = BPW * C
K = 48
NCHUNK = RPW // K


def _sc_body(x_hbm, idx_hbm, out_hbm, idx_v, adj_v, buf0, buf1,
             gsem0, gsem1, wsem0, wsem1):
    wid = lax.axis_index("s") * NC + lax.axis_index("c")
    out_base = wid * RPW
    pltpu.sync_copy(idx_hbm, idx_v)
    for bi in range(BPW):
        row0 = (wid * BPW + bi) * C
        for t in range(C // L):
            sl_in = pl.ds(t * L, L)
            sl_out = pl.ds(bi * C + t * L, L)
            adj_v[sl_out] = idx_v[sl_in] + row0

    bufs = (buf0, buf1)
    gsems = (gsem0, gsem1)
    wsems = (wsem0, wsem1)

    def fire(j):
        return pltpu.async_copy(
            x_hbm.at[adj_v.at[pl.ds(j * K, K)]], bufs[j % 2], gsems[j % 2]
        )

    gc = [fire(0), None]
    wc = [None, None]
    for j in range(NCHUNK):
        p = j % 2
        q = p ^ 1
        if j + 1 < NCHUNK:
            if wc[q] is not None:
                wc[q].wait()
                wc[q] = None
            gc[q] = fire(j + 1)
        gc[p].wait()
        wc[p] = pltpu.async_copy(
            bufs[p], out_hbm.at[pl.ds(out_base + j * K, K)], wsems[p]
        )
    for p in range(2):
        if wc[p] is not None:
            wc[p].wait()


@jax.jit
def _sc_gather(x2, idx):
    mesh = plsc.VectorSubcoreMesh(core_axis_name="c", subcore_axis_name="s")
    return pl.kernel(
        _sc_body,
        mesh=mesh,
        out_type=jax.ShapeDtypeStruct((B * C, D), jnp.float32),
        scratch_types=[
            pltpu.VMEM((C,), jnp.int32),
            pltpu.VMEM((RPW,), jnp.int32),
            pltpu.VMEM((K, D), jnp.float32),
            pltpu.VMEM((K, D), jnp.float32),
            pltpu.SemaphoreType.DMA,
            pltpu.SemaphoreType.DMA,
            pltpu.SemaphoreType.DMA,
            pltpu.SemaphoreType.DMA,
        ],
    )(x2, idx)


def kernel(x, indices):
    idx = indices.astype(jnp.int32)
    out2 = _sc_gather(x.reshape(B * C, D), idx)
    return out2.reshape(B, C, H, W)


# speedup vs baseline: 5.0794x; 2.9769x over previous
import jax
import jax.numpy as jnp
from jax.experimental import pallas as pl
from jax.experimental.pallas import tpu as pltpu

B, C, H, W = 64, 384, 32, 32
NROW = B * H * W
RB = 2048


def _body(x_ref, idx_ref, o_ref):
    idx = idx_ref[...]
    off2d = jnp.broadcast_to((idx % 128)[None, :], (RB, C))
    sel = idx // 128
    x = x_ref[...]
    g0 = jnp.take_along_axis(x[:, 0:128], off2d, axis=1)
    g1 = jnp.take_along_axis(x[:, 128:256], off2d, axis=1)
    g2 = jnp.take_along_axis(x[:, 256:384], off2d, axis=1)
    sel2d = jnp.broadcast_to(sel[None, :], (RB, C))
    o_ref[...] = jnp.where(sel2d == 0, g0, jnp.where(sel2d == 1, g1, g2))


@jax.jit
def _tc_gather(xT, idx):
    return pl.pallas_call(
        _body,
        grid=(NROW // RB,),
        in_specs=[
            pl.BlockSpec((RB, C), lambda i: (i, 0)),
            pl.BlockSpec((C,), lambda i: (0,)),
        ],
        out_specs=pl.BlockSpec((RB, C), lambda i: (i, 0)),
        out_shape=jax.ShapeDtypeStruct((NROW, C), jnp.float32),
    )(xT, idx)


def kernel(x, indices):
    idx = indices.astype(jnp.int32)
    xT = x.transpose(0, 2, 3, 1).reshape(NROW, C)
    out2 = _tc_gather(xT, idx)
    return out2.reshape(B, H, W, C).transpose(0, 3, 1, 2)
